# Initial kernel scaffold; baseline (speedup 1.0000x reference)
import jax
import jax.numpy as jnp
from jax.experimental import pallas as pl


def kernel(features, edge_index, W_self1, W_neigh1, b1, W_self2, W_neigh2, b2):
    raise NotImplementedError("write your pallas kernel here")


# speedup vs baseline: 4.9346x; 4.9346x over previous
import functools

import jax
import jax.numpy as jnp
from jax import lax
from jax.experimental import pallas as pl
from jax.experimental.pallas import tpu as pltpu
from jax.experimental.pallas import tpu_sc as plsc

N = 10000
E = 320000
F = 128
H = 256

NPAD = 10240
RPT = NPAD // 16
EPAD = 323584
EPT = EPAD // 32
CH = 128
NCHUNK = EPT // CH


def _make_segsum(with_deg):
    mesh = plsc.VectorSubcoreMesh(core_axis_name="c", subcore_axis_name="s")
    out_type = [jax.ShapeDtypeStruct((2, NPAD, F), jnp.float32)]
    if with_deg:
        out_type.append(jax.ShapeDtypeStruct((2 * NPAD,), jnp.float32))
    scratch = [
        pltpu.VMEM((CH,), jnp.int32),
        pltpu.VMEM((CH,), jnp.int32),
        pltpu.VMEM((CH, F), jnp.float32),
        pltpu.VMEM((CH,), jnp.float32),
        pltpu.VMEM((RPT,), jnp.float32),
        pltpu.VMEM_SHARED((NPAD, F), jnp.float32),
        pltpu.VMEM_SHARED((NPAD,), jnp.float32),
        pltpu.SemaphoreType.DMA,
    ]

    @functools.partial(pl.kernel, mesh=mesh, out_type=out_type,
                       scratch_types=scratch)
    def seg(x_hbm, src_hbm, dst_hbm, zrows_hbm, *rest):
        if with_deg:
            out_hbm, deg_hbm = rest[0], rest[1]
            srcv, dstv, rows, onesv, dv, acc, dacc, sem = rest[2:]
        else:
            out_hbm = rest[0]
            srcv, dstv, rows, onesv, dv, acc, dacc, sem = rest[1:]
        c = lax.axis_index("c")
        s = lax.axis_index("s")
        wid = c * 16 + s

        pltpu.sync_copy(zrows_hbm, acc.at[pl.ds(s * RPT, RPT)])
        if with_deg:
            for k in range(RPT // 16):
                dv[pl.ds(k * 16, 16)] = jnp.zeros((16,), jnp.float32)
            pltpu.sync_copy(dv, dacc.at[pl.ds(s * RPT, RPT)])
            for k in range(CH // 16):
                onesv[pl.ds(k * 16, 16)] = jnp.ones((16,), jnp.float32)
        plsc.subcore_barrier()

        def chunk(j, carry):
            b = pl.multiple_of(wid * EPT + j * CH, CH)
            pltpu.sync_copy(src_hbm.at[pl.ds(b, CH)], srcv)
            pltpu.sync_copy(dst_hbm.at[pl.ds(b, CH)], dstv)
            pltpu.async_copy(x_hbm.at[srcv], rows, sem).wait()
            pltpu.sync_copy(rows, acc.at[dstv], add=True)
            if with_deg:
                pltpu.sync_copy(onesv, dacc.at[dstv], add=True)
            return carry

        lax.fori_loop(0, NCHUNK, chunk, 0)
        plsc.subcore_barrier()

        pltpu.sync_copy(acc.at[pl.ds(s * RPT, RPT)],
                        out_hbm.at[c, pl.ds(s * RPT, RPT)])
        if with_deg:
            pltpu.sync_copy(dacc.at[pl.ds(s * RPT, RPT)], dv)
            pltpu.sync_copy(dv, deg_hbm.at[pl.ds(c * NPAD + s * RPT, RPT)])

    return seg


_segsum_deg = _make_segsum(True)
_segsum = _make_segsum(False)

R = 1000
GRID = N // R


def _tc1_body(x_r, p1a_r, p1b_r, degt_r, ws1_r, wn1_r, b1_r, wn2_r, ws2_r,
              b2_r, hs_r, y_r, rdegb_r):
    deg = degt_r[:, 0] + degt_r[:, 1]
    rdeg = 1.0 / jnp.maximum(deg, 1.0)
    mean1 = (p1a_r[...] + p1b_r[...]) * rdeg[:, None]
    h = x_r[...] @ ws1_r[...] + mean1 @ wn1_r[...] + b1_r[...]
    h = jnp.maximum(h, 0.0)
    hs_r[...] = h @ ws2_r[...] + b2_r[...]
    y_r[...] = h @ wn2_r[...]
    rdegb_r[...] = jnp.broadcast_to(rdeg[:, None], (R, F))


def _tc1(x, p1a, p1b, degt, ws1, wn1, b1, wn2, ws2, b2):
    return pl.pallas_call(
        _tc1_body,
        grid=(GRID,),
        in_specs=[
            pl.BlockSpec((R, F), lambda i: (i, 0)),
            pl.BlockSpec((R, F), lambda i: (i, 0)),
            pl.BlockSpec((R, F), lambda i: (i, 0)),
            pl.BlockSpec((R, 2), lambda i: (i, 0)),
            pl.BlockSpec((F, H), lambda i: (0, 0)),
            pl.BlockSpec((F, H), lambda i: (0, 0)),
            pl.BlockSpec((1, H), lambda i: (0, 0)),
            pl.BlockSpec((H, F), lambda i: (0, 0)),
            pl.BlockSpec((H, F), lambda i: (0, 0)),
            pl.BlockSpec((1, F), lambda i: (0, 0)),
        ],
        out_specs=[
            pl.BlockSpec((R, F), lambda i: (i, 0)),
            pl.BlockSpec((R, F), lambda i: (i, 0)),
            pl.BlockSpec((R, F), lambda i: (i, 0)),
        ],
        out_shape=[
            jax.ShapeDtypeStruct((N, F), jnp.float32),
            jax.ShapeDtypeStruct((N, F), jnp.float32),
            jax.ShapeDtypeStruct((N, F), jnp.float32),
        ],
    )(x, p1a, p1b, degt, ws1, wn1, b1, wn2, ws2, b2)


def _tc2_body(hs_r, p2a_r, p2b_r, rdegb_r, out_r):
    out_r[...] = hs_r[...] + (p2a_r[...] + p2b_r[...]) * rdegb_r[...]


def _tc2(hs, p2a, p2b, rdegb):
    return pl.pallas_call(
        _tc2_body,
        grid=(GRID,),
        in_specs=[pl.BlockSpec((R, F), lambda i: (i, 0))] * 4,
        out_specs=pl.BlockSpec((R, F), lambda i: (i, 0)),
        out_shape=jax.ShapeDtypeStruct((N, F), jnp.float32),
    )(hs, p2a, p2b, rdegb)


def kernel(features, edge_index, W_self1, W_neigh1, b1, W_self2, W_neigh2, b2):
    src = edge_index[0].astype(jnp.int32)
    dst = edge_index[1].astype(jnp.int32)
    pad = EPAD - E
    src_p = jnp.concatenate([src, jnp.zeros((pad,), jnp.int32)])
    dst_p = jnp.concatenate([dst, jnp.full((pad,), N, jnp.int32)])
    zrows = jnp.zeros((RPT, F), jnp.float32)

    p1, pdeg = _segsum_deg(features, src_p, dst_p, zrows)
    p1a = p1[0, :N]
    p1b = p1[1, :N]
    degt = jnp.transpose(pdeg.reshape(2, NPAD)[:, :N])

    hs, y, rdegb = _tc1(features, p1a, p1b, degt, W_self1, W_neigh1,
                        b1.reshape(1, H), W_neigh2, W_self2,
                        b2.reshape(1, F))

    p2 = _segsum(y, src_p, dst_p, zrows)
    if isinstance(p2, (list, tuple)):
        p2 = p2[0]
    out = _tc2(hs, p2[0, :N], p2[1, :N], rdegb)
    return out
